# Initial kernel scaffold; baseline (speedup 1.0000x reference)
import jax
import jax.numpy as jnp
from jax.experimental import pallas as pl


def kernel(x, edge_index, adj_values, weight):
    raise NotImplementedError("write your pallas kernel here")


# speedup vs baseline: 2.2205x; 2.2205x over previous
import functools

import jax
import jax.numpy as jnp
from jax import lax
from jax.experimental import pallas as pl
from jax.experimental.pallas import tpu as pltpu
from jax.experimental.pallas import tpu_sc as plsc

N_NODES = 10000
N_EDGES = 160000
IN_SIZE = 256
OUT_SIZE = 256

NC = 2
NS = 16
HALF = OUT_SIZE // NC
CHUNK = 128
EPT = 10240
E_PAD = EPT * NS
N_PAD = 10240
ROWS_PER_SUB = N_PAD // NS

BN = 1000


def _matmul_block(x_ref, w_ref, o_ref):
    o_ref[0, :, :] = jnp.dot(x_ref[...], w_ref[...],
                             preferred_element_type=jnp.float32)


def _support_halves(x, weight):
    return pl.pallas_call(
        _matmul_block,
        grid=(NC, N_NODES // BN),
        in_specs=[
            pl.BlockSpec((BN, IN_SIZE), lambda c, i: (i, 0)),
            pl.BlockSpec((IN_SIZE, HALF), lambda c, i: (0, c)),
        ],
        out_specs=pl.BlockSpec((1, BN, HALF), lambda c, i: (c, i, 0)),
        out_shape=jax.ShapeDtypeStruct((NC, N_NODES, HALF), jnp.float32),
    )(x, weight)


def _sc_body(sup_ref, col_ref, row_ref, adj_ref, out_ref,
             colb, rowb, adjb, buf, acc, sem):
    c = lax.axis_index("c")
    s = lax.axis_index("s")

    def _zero_row(i, _):
        for j in range(HALF // 16):
            buf[i, pl.ds(j * 16, 16)] = jnp.zeros((16,), jnp.float32)
        return _
    lax.fori_loop(0, CHUNK, _zero_row, None)
    for k in range(ROWS_PER_SUB // CHUNK):
        pltpu.sync_copy(buf, acc.at[pl.ds(s * ROWS_PER_SUB + k * CHUNK, CHUNK)])
    plsc.subcore_barrier()

    def _chunk(q, _):
        base = s * EPT + q * CHUNK
        pltpu.sync_copy(col_ref.at[pl.ds(base, CHUNK)], colb)
        pltpu.sync_copy(row_ref.at[pl.ds(base, CHUNK)], rowb)
        pltpu.sync_copy(adj_ref.at[pl.ds(base, CHUNK)], adjb)
        pltpu.async_copy(sup_ref.at[c].at[colb], buf, sem).wait()

        def _scale(g, _):
            av = adjb[pl.ds(g * 16, 16)]
            for l in range(16):
                a = av[l]
                e = g * 16 + l
                for j in range(HALF // 16):
                    sl = pl.ds(j * 16, 16)
                    buf[e, sl] = buf[e, sl] * a
            return _
        lax.fori_loop(0, CHUNK // 16, _scale, None)
        pltpu.sync_copy(buf, acc.at[rowb], add=True)
        return _
    lax.fori_loop(0, EPT // CHUNK, _chunk, None)

    plsc.subcore_barrier()
    pltpu.sync_copy(acc.at[pl.ds(s * ROWS_PER_SUB, ROWS_PER_SUB)],
                    out_ref.at[c].at[pl.ds(s * ROWS_PER_SUB, ROWS_PER_SUB)])


def _sc_scatter(support_t, col_p, row_p, adj_p):
    mesh = plsc.VectorSubcoreMesh(core_axis_name="c", subcore_axis_name="s")
    k = pl.kernel(
        _sc_body,
        out_type=jax.ShapeDtypeStruct((NC, N_PAD, HALF), jnp.float32),
        mesh=mesh,
        scratch_types=[
            pltpu.VMEM((CHUNK,), jnp.int32),
            pltpu.VMEM((CHUNK,), jnp.int32),
            pltpu.VMEM((CHUNK,), jnp.float32),
            pltpu.VMEM((CHUNK, HALF), jnp.float32),
            pltpu.VMEM_SHARED((N_PAD, HALF), jnp.float32),
            pltpu.SemaphoreType.DMA,
        ],
    )
    return k(support_t, col_p, row_p, adj_p)


def kernel(x, edge_index, adj_values, weight):
    ei = edge_index.astype(jnp.int32)
    row_p = jnp.zeros((E_PAD,), jnp.int32).at[:N_EDGES].set(ei[0])
    col_p = jnp.zeros((E_PAD,), jnp.int32).at[:N_EDGES].set(ei[1])
    adj_p = jnp.zeros((E_PAD,), jnp.float32).at[:N_EDGES].set(adj_values)
    support_t = _support_halves(x, weight)
    out2 = _sc_scatter(support_t, col_p, row_p, adj_p)
    return out2[:, :N_NODES, :].transpose(1, 0, 2).reshape(N_NODES, OUT_SIZE)


# speedup vs baseline: 3.2028x; 1.4424x over previous
import functools

import jax
import jax.numpy as jnp
from jax import lax
from jax.experimental import pallas as pl
from jax.experimental.pallas import tpu as pltpu
from jax.experimental.pallas import tpu_sc as plsc

N_NODES = 10000
N_EDGES = 160000
IN_SIZE = 256
OUT_SIZE = 256

NC = 2
NS = 16
HALF = OUT_SIZE // NC
CHUNK = 64
EPT = 10240
E_PAD = EPT * NS
NQ = EPT // CHUNK
N_PAD = 10240
ROWS_PER_SUB = N_PAD // NS

BN = 1000


def _matmul_block(x_ref, w_ref, o_ref):
    o_ref[0, :, :] = jnp.dot(x_ref[...], w_ref[...],
                             preferred_element_type=jnp.float32)


def _support_halves(x, weight):
    return pl.pallas_call(
        _matmul_block,
        grid=(NC, N_NODES // BN),
        in_specs=[
            pl.BlockSpec((BN, IN_SIZE), lambda c, i: (i, 0)),
            pl.BlockSpec((IN_SIZE, HALF), lambda c, i: (0, c)),
        ],
        out_specs=pl.BlockSpec((1, BN, HALF), lambda c, i: (c, i, 0)),
        out_shape=jax.ShapeDtypeStruct((NC, N_NODES, HALF), jnp.float32),
    )(x, weight)


def _sc_body(sup_ref, col_ref, row_ref, adj_ref, out_ref,
             colb, rowb, adjb, gbufs, sbufs, acc, isems, gsems, ssems):
    c = lax.axis_index("c")
    s = lax.axis_index("s")
    sup = sup_ref.at[c]
    hbm_dummy = sup.at[pl.ds(0, CHUNK)]
    ebase = s * EPT

    def _issue_idx(q, k4):
        sl = pl.ds(ebase + q * CHUNK, CHUNK)
        pltpu.async_copy(col_ref.at[sl], colb[k4], isems[k4])
        pltpu.async_copy(row_ref.at[sl], rowb[k4], isems[k4])
        pltpu.async_copy(adj_ref.at[sl], adjb[k4], isems[k4])

    idummy = pl.ds(0, CHUNK)

    def _wait_idx(k4):
        pltpu.make_async_copy(col_ref.at[idummy], colb[k4], isems[k4]).wait()
        pltpu.make_async_copy(row_ref.at[idummy], rowb[k4], isems[k4]).wait()
        pltpu.make_async_copy(adj_ref.at[idummy], adjb[k4], isems[k4]).wait()

    def _zero_row(i, carry):
        for j in range(HALF // 16):
            sbufs[0][i, pl.ds(j * 16, 16)] = jnp.zeros((16,), jnp.float32)
        return carry
    lax.fori_loop(0, CHUNK, _zero_row, None)
    for k in range(ROWS_PER_SUB // CHUNK):
        pltpu.sync_copy(sbufs[0],
                        acc.at[pl.ds(s * ROWS_PER_SUB + k * CHUNK, CHUNK)])

    for q in range(4):
        _issue_idx(q, q)
    for q in range(2):
        _wait_idx(q)
        pltpu.async_copy(sup.at[colb[q]], gbufs[q], gsems[q])
    plsc.subcore_barrier()

    def _visit(t, b4):
        q = 4 * t + b4
        b2 = b4 % 2
        k4 = (b4 + 2) % 4
        gbuf, sbuf = gbufs[b2], sbufs[b2]
        pltpu.make_async_copy(hbm_dummy, gbuf, gsems[b2]).wait()

        @pl.when(q >= 2)
        def _drain():
            pltpu.make_async_copy(hbm_dummy, sbuf, ssems[b2]).wait()

        @pl.when((q >= 2) & (q + 2 < NQ))
        def _refill_idx():
            _issue_idx(q + 2, k4)

        def _scale(g, carry):
            av = adjb[b4][pl.ds(g * 16, 16)]
            for l in range(16):
                a = av[l]
                e = g * 16 + l
                for j in range(HALF // 16):
                    sl = pl.ds(j * 16, 16)
                    sbuf[e, sl] = gbuf[e, sl] * a
            return carry
        lax.fori_loop(0, CHUNK // 16, _scale, None)

        pltpu.async_copy(sbuf, acc.at[rowb[b4]], ssems[b2], add=True)

        @pl.when(q + 2 < NQ)
        def _refill_gather():
            _wait_idx(k4)
            pltpu.async_copy(sup.at[colb[k4]], gbuf, gsems[b2])

    def _step(t, carry):
        for b4 in range(4):
            _visit(t, b4)
        return carry
    lax.fori_loop(0, NQ // 4, _step, None)

    pltpu.make_async_copy(hbm_dummy, sbufs[0], ssems[0]).wait()
    pltpu.make_async_copy(hbm_dummy, sbufs[1], ssems[1]).wait()
    plsc.subcore_barrier()
    pltpu.sync_copy(acc.at[pl.ds(s * ROWS_PER_SUB, ROWS_PER_SUB)],
                    out_ref.at[c].at[pl.ds(s * ROWS_PER_SUB, ROWS_PER_SUB)])


def _sc_scatter(support_t, col_p, row_p, adj_p):
    mesh = plsc.VectorSubcoreMesh(core_axis_name="c", subcore_axis_name="s")
    k = pl.kernel(
        lambda sup, col, row, adj, out, *scr: _sc_body(
            sup, col, row, adj, out,
            [scr[0], scr[1], scr[2], scr[3]],
            [scr[4], scr[5], scr[6], scr[7]],
            [scr[8], scr[9], scr[10], scr[11]],
            [scr[12], scr[13]],
            [scr[14], scr[15]],
            scr[16],
            [scr[17], scr[18], scr[19], scr[20]],
            [scr[21], scr[22]],
            [scr[23], scr[24]],
        ),
        out_type=jax.ShapeDtypeStruct((NC, N_PAD, HALF), jnp.float32),
        mesh=mesh,
        scratch_types=(
            [pltpu.VMEM((CHUNK,), jnp.int32) for _ in range(4)]
            + [pltpu.VMEM((CHUNK,), jnp.int32) for _ in range(4)]
            + [pltpu.VMEM((CHUNK,), jnp.float32) for _ in range(4)]
            + [pltpu.VMEM((CHUNK, HALF), jnp.float32) for _ in range(4)]
            + [pltpu.VMEM_SHARED((N_PAD, HALF), jnp.float32)]
            + [pltpu.SemaphoreType.DMA for _ in range(8)]
        ),
    )
    return k(support_t, col_p, row_p, adj_p)


def kernel(x, edge_index, adj_values, weight):
    ei = edge_index.astype(jnp.int32)
    row_p = jnp.zeros((E_PAD,), jnp.int32).at[:N_EDGES].set(ei[0])
    col_p = jnp.zeros((E_PAD,), jnp.int32).at[:N_EDGES].set(ei[1])
    adj_p = jnp.zeros((E_PAD,), jnp.float32).at[:N_EDGES].set(adj_values)
    support_t = _support_halves(x, weight)
    out2 = _sc_scatter(support_t, col_p, row_p, adj_p)
    return out2[:, :N_NODES, :].transpose(1, 0, 2).reshape(N_NODES, OUT_SIZE)
